# initial kernel scaffold (unmeasured)
import jax
import jax.numpy as jnp
from jax import lax
from jax.experimental import pallas as pl
from jax.experimental.pallas import tpu as pltpu

N_LOCAL = 4
T = 2048
D = 1024
F = 2048
CHUNK = 512
N_CHUNKS = T // CHUNK


def kernel(x, assign, W1, W2):
    xb = x.astype(jnp.bfloat16)
    w1b = W1.astype(jnp.bfloat16)
    w2b = W2.astype(jnp.bfloat16)

    my_x = lax.axis_index("x")
    loc_cols = my_x * N_LOCAL + jnp.arange(N_LOCAL, dtype=jnp.int32)
    rem_cols = (1 - my_x) * N_LOCAL + jnp.arange(N_LOCAL, dtype=jnp.int32)
    mask_loc = (assign[:, None] == loc_cols[None, :]).astype(jnp.float32)
    mask_rem = (assign[:, None] == rem_cols[None, :]).astype(jnp.float32)

    def body(x_ref, mloc_ref, mrem_ref, w1_ref, w2_ref, out_ref,
             rx, rmask, pacc, rback, w1v, w2v,
             w_sems, sx_send, sx_recv, sm_send, sm_recv, sb_send, sb_recv):
        mx = lax.axis_index("x")
        my = lax.axis_index("y")
        mz = lax.axis_index("z")
        partner = (1 - mx, my, mz)

        barrier = pltpu.get_barrier_semaphore()
        pl.semaphore_signal(barrier, inc=1, device_id=partner,
                            device_id_type=pl.DeviceIdType.MESH)
        pl.semaphore_wait(barrier, 1)

        rd_x = pltpu.make_async_remote_copy(
            src_ref=x_ref, dst_ref=rx, send_sem=sx_send, recv_sem=sx_recv,
            device_id=partner, device_id_type=pl.DeviceIdType.MESH)
        rd_m = pltpu.make_async_remote_copy(
            src_ref=mrem_ref, dst_ref=rmask, send_sem=sm_send,
            recv_sem=sm_recv,
            device_id=partner, device_id_type=pl.DeviceIdType.MESH)
        rd_x.start()
        rd_m.start()

        pltpu.make_async_copy(w1_ref.at[0], w1v.at[0], w_sems.at[0, 0]).start()
        pltpu.make_async_copy(w2_ref.at[0], w2v.at[0], w_sems.at[0, 1]).start()

        rd_x.wait()
        rd_m.wait()

        out_ref[...] = jnp.zeros_like(out_ref)
        pacc[...] = jnp.zeros_like(pacc)

        for e in range(N_LOCAL):
            slot = e % 2
            pltpu.make_async_copy(
                w1_ref.at[e], w1v.at[slot], w_sems.at[slot, 0]).wait()
            pltpu.make_async_copy(
                w2_ref.at[e], w2v.at[slot], w_sems.at[slot, 1]).wait()
            if e + 1 < N_LOCAL:
                ns = (e + 1) % 2
                pltpu.make_async_copy(
                    w1_ref.at[e + 1], w1v.at[ns], w_sems.at[ns, 0]).start()
                pltpu.make_async_copy(
                    w2_ref.at[e + 1], w2v.at[ns], w_sems.at[ns, 1]).start()
            w1 = w1v[slot]
            w2 = w2v[slot]
            for c in range(N_CHUNKS):
                sl = pl.ds(c * CHUNK, CHUNK)
                xc = x_ref[sl, :]
                h = jnp.dot(xc, w1, preferred_element_type=jnp.float32)
                h = jnp.maximum(h, 0.0).astype(jnp.bfloat16)
                o = jnp.dot(h, w2, preferred_element_type=jnp.float32)
                m = mloc_ref[sl, e:e + 1]
                out_ref[sl, :] = out_ref[sl, :] + m * o
                xp = rx[sl, :]
                hp = jnp.dot(xp, w1, preferred_element_type=jnp.float32)
                hp = jnp.maximum(hp, 0.0).astype(jnp.bfloat16)
                op = jnp.dot(hp, w2, preferred_element_type=jnp.float32)
                mp = rmask[sl, e:e + 1]
                pacc[sl, :] = pacc[sl, :] + (mp * op).astype(jnp.bfloat16)

        rd_b = pltpu.make_async_remote_copy(
            src_ref=pacc, dst_ref=rback, send_sem=sb_send, recv_sem=sb_recv,
            device_id=partner, device_id_type=pl.DeviceIdType.MESH)
        rd_b.start()
        rd_b.wait()

        out_ref[...] = out_ref[...] + rback[...].astype(jnp.float32)

    return pl.pallas_call(
        body,
        out_shape=jax.ShapeDtypeStruct((T, D), jnp.float32),
        in_specs=[
            pl.BlockSpec(memory_space=pltpu.VMEM),
            pl.BlockSpec(memory_space=pltpu.VMEM),
            pl.BlockSpec(memory_space=pltpu.VMEM),
            pl.BlockSpec(memory_space=pltpu.ANY),
            pl.BlockSpec(memory_space=pltpu.ANY),
        ],
        out_specs=pl.BlockSpec(memory_space=pltpu.VMEM),
        scratch_shapes=[
            pltpu.VMEM((T, D), jnp.bfloat16),
            pltpu.VMEM((T, N_LOCAL), jnp.float32),
            pltpu.VMEM((T, D), jnp.bfloat16),
            pltpu.VMEM((T, D), jnp.bfloat16),
            pltpu.VMEM((2, D, F), jnp.bfloat16),
            pltpu.VMEM((2, F, D), jnp.bfloat16),
            pltpu.SemaphoreType.DMA((2, 2)),
            pltpu.SemaphoreType.DMA,
            pltpu.SemaphoreType.DMA,
            pltpu.SemaphoreType.DMA,
            pltpu.SemaphoreType.DMA,
            pltpu.SemaphoreType.DMA,
            pltpu.SemaphoreType.DMA,
        ],
        compiler_params=pltpu.CompilerParams(collective_id=0),
    )(xb, mask_loc, mask_rem, w1b, w2b)


# baseline (device time: 306471 ns/iter reference)
import jax
import jax.numpy as jnp
from jax import lax
from jax.experimental import pallas as pl
from jax.experimental.pallas import tpu as pltpu

N_LOCAL = 4
T = 2048
D = 1024
F = 2048
CHUNK = 512
N_CHUNKS = T // CHUNK


def kernel(x, assign, W1, W2):
    xb = x.astype(jnp.bfloat16)
    w1b = W1.astype(jnp.bfloat16)
    w2b = W2.astype(jnp.bfloat16)

    my_x = lax.axis_index("x")
    loc_cols = my_x * N_LOCAL + jnp.arange(N_LOCAL, dtype=jnp.int32)
    rem_cols = (1 - my_x) * N_LOCAL + jnp.arange(N_LOCAL, dtype=jnp.int32)
    mask_loc = (assign[:, None] == loc_cols[None, :]).astype(jnp.float32)
    mask_rem = (assign[:, None] == rem_cols[None, :]).astype(jnp.float32)

    def body(x_ref, mloc_ref, mrem_ref, w1_ref, w2_ref, out_ref,
             rx, rmask, pacc, rback, w1v, w2v,
             w_sems, sx_send, sx_recv, sm_send, sm_recv, sb_send, sb_recv):
        mx = lax.axis_index("x")
        my = lax.axis_index("y")
        mz = lax.axis_index("z")
        partner = (1 - mx, my, mz)

        barrier = pltpu.get_barrier_semaphore()
        pl.semaphore_signal(barrier, inc=1, device_id=partner,
                            device_id_type=pl.DeviceIdType.MESH)
        pl.semaphore_wait(barrier, 1)

        rd_x = pltpu.make_async_remote_copy(
            src_ref=x_ref, dst_ref=rx, send_sem=sx_send, recv_sem=sx_recv,
            device_id=partner, device_id_type=pl.DeviceIdType.MESH)
        rd_m = pltpu.make_async_remote_copy(
            src_ref=mrem_ref, dst_ref=rmask, send_sem=sm_send,
            recv_sem=sm_recv,
            device_id=partner, device_id_type=pl.DeviceIdType.MESH)
        rd_x.start()
        rd_m.start()

        pltpu.make_async_copy(w1_ref.at[0], w1v.at[0], w_sems.at[0, 0]).start()
        pltpu.make_async_copy(w2_ref.at[0], w2v.at[0], w_sems.at[0, 1]).start()

        rd_x.wait()
        rd_m.wait()

        out_ref[...] = jnp.zeros_like(out_ref)
        pacc[...] = jnp.zeros_like(pacc)

        for e in range(N_LOCAL):
            slot = e % 2
            pltpu.make_async_copy(
                w1_ref.at[e], w1v.at[slot], w_sems.at[slot, 0]).wait()
            pltpu.make_async_copy(
                w2_ref.at[e], w2v.at[slot], w_sems.at[slot, 1]).wait()
            if e + 1 < N_LOCAL:
                ns = (e + 1) % 2
                pltpu.make_async_copy(
                    w1_ref.at[e + 1], w1v.at[ns], w_sems.at[ns, 0]).start()
                pltpu.make_async_copy(
                    w2_ref.at[e + 1], w2v.at[ns], w_sems.at[ns, 1]).start()
            w1 = w1v[slot]
            w2 = w2v[slot]
            for c in range(N_CHUNKS):
                sl = pl.ds(c * CHUNK, CHUNK)
                xc = x_ref[sl, :]
                h = jnp.dot(xc, w1, preferred_element_type=jnp.float32)
                h = jnp.maximum(h, 0.0).astype(jnp.bfloat16)
                o = jnp.dot(h, w2, preferred_element_type=jnp.float32)
                m = mloc_ref[sl, e:e + 1]
                out_ref[sl, :] = out_ref[sl, :] + m * o
                xp = rx[sl, :]
                hp = jnp.dot(xp, w1, preferred_element_type=jnp.float32)
                hp = jnp.maximum(hp, 0.0).astype(jnp.bfloat16)
                op = jnp.dot(hp, w2, preferred_element_type=jnp.float32)
                mp = rmask[sl, e:e + 1]
                pacc[sl, :] = pacc[sl, :] + (mp * op).astype(jnp.bfloat16)

        rd_b = pltpu.make_async_remote_copy(
            src_ref=pacc, dst_ref=rback, send_sem=sb_send, recv_sem=sb_recv,
            device_id=partner, device_id_type=pl.DeviceIdType.MESH)
        rd_b.start()
        rd_b.wait()

        out_ref[...] = out_ref[...] + rback[...].astype(jnp.float32)

    return pl.pallas_call(
        body,
        out_shape=jax.ShapeDtypeStruct((T, D), jnp.float32),
        in_specs=[
            pl.BlockSpec(memory_space=pltpu.VMEM),
            pl.BlockSpec(memory_space=pltpu.VMEM),
            pl.BlockSpec(memory_space=pltpu.VMEM),
            pl.BlockSpec(memory_space=pltpu.HBM),
            pl.BlockSpec(memory_space=pltpu.HBM),
        ],
        out_specs=pl.BlockSpec(memory_space=pltpu.VMEM),
        scratch_shapes=[
            pltpu.VMEM((T, D), jnp.bfloat16),
            pltpu.VMEM((T, N_LOCAL), jnp.float32),
            pltpu.VMEM((T, D), jnp.bfloat16),
            pltpu.VMEM((T, D), jnp.bfloat16),
            pltpu.VMEM((2, D, F), jnp.bfloat16),
            pltpu.VMEM((2, F, D), jnp.bfloat16),
            pltpu.SemaphoreType.DMA((2, 2)),
            pltpu.SemaphoreType.DMA,
            pltpu.SemaphoreType.DMA,
            pltpu.SemaphoreType.DMA,
            pltpu.SemaphoreType.DMA,
            pltpu.SemaphoreType.DMA,
            pltpu.SemaphoreType.DMA,
        ],
        compiler_params=pltpu.CompilerParams(
            collective_id=0, vmem_limit_bytes=56 * 1024 * 1024),
    )(xb, mask_loc, mask_rem, w1b, w2b)


# device time: 220539 ns/iter; 1.3896x vs baseline; 1.3896x over previous
import jax
import jax.numpy as jnp
from jax import lax
from jax.experimental import pallas as pl
from jax.experimental.pallas import tpu as pltpu

N_LOCAL = 4
T = 2048
D = 1024
F = 2048
CHUNK = 512
N_CHUNKS = T // CHUNK


def kernel(x, assign, W1, W2):
    xb = x.astype(jnp.bfloat16)
    w1b = W1.astype(jnp.bfloat16)
    w2b = W2.astype(jnp.bfloat16)

    my_x = lax.axis_index("x")
    loc_cols = my_x * N_LOCAL + jnp.arange(N_LOCAL, dtype=jnp.int32)
    rem_cols = (1 - my_x) * N_LOCAL + jnp.arange(N_LOCAL, dtype=jnp.int32)
    mask_loc = (assign[:, None] == loc_cols[None, :]).astype(jnp.float32)
    mask_rem = (assign[:, None] == rem_cols[None, :]).astype(jnp.float32)

    def body(x_ref, mloc_ref, mrem_ref, w1_ref, w2_ref, out_ref,
             rx, rmask, pacc, rback, w1v, w2v,
             w_sems, sx_send, sx_recv, sm_send, sm_recv, sb_send, sb_recv):
        mx = lax.axis_index("x")
        my = lax.axis_index("y")
        mz = lax.axis_index("z")
        partner = (1 - mx, my, mz)

        def start_load(e):
            slot = e % 2
            pltpu.make_async_copy(w1_ref.at[e], w1v.at[slot],
                                  w_sems.at[slot, 0]).start()
            pltpu.make_async_copy(w2_ref.at[e], w2v.at[slot],
                                  w_sems.at[slot, 1]).start()

        def wait_load(e):
            slot = e % 2
            pltpu.make_async_copy(w1_ref.at[e], w1v.at[slot],
                                  w_sems.at[slot, 0]).wait()
            pltpu.make_async_copy(w2_ref.at[e], w2v.at[slot],
                                  w_sems.at[slot, 1]).wait()

        def ffn_chunk(src_ref, sl, slot):
            h = jnp.dot(src_ref[sl, :], w1v[slot],
                        preferred_element_type=jnp.float32)
            h = jnp.maximum(h, 0.0).astype(jnp.bfloat16)
            return jnp.dot(h, w2v[slot], preferred_element_type=jnp.float32)

        def back_rdma(c):
            sl = pl.ds(c * CHUNK, CHUNK)
            return pltpu.make_async_remote_copy(
                src_ref=pacc.at[sl], dst_ref=rback.at[sl],
                send_sem=sb_send.at[c], recv_sem=sb_recv.at[c],
                device_id=partner, device_id_type=pl.DeviceIdType.MESH)

        barrier = pltpu.get_barrier_semaphore()
        pl.semaphore_signal(barrier, inc=1, device_id=partner,
                            device_id_type=pl.DeviceIdType.MESH)
        pl.semaphore_wait(barrier, 1)

        rd_x = pltpu.make_async_remote_copy(
            src_ref=x_ref, dst_ref=rx, send_sem=sx_send, recv_sem=sx_recv,
            device_id=partner, device_id_type=pl.DeviceIdType.MESH)
        rd_m = pltpu.make_async_remote_copy(
            src_ref=mrem_ref, dst_ref=rmask, send_sem=sm_send,
            recv_sem=sm_recv,
            device_id=partner, device_id_type=pl.DeviceIdType.MESH)
        rd_x.start()
        rd_m.start()

        start_load(0)
        start_load(1)

        out_ref[...] = jnp.zeros_like(out_ref)
        for e in range(N_LOCAL):
            wait_load(e)
            slot = e % 2

            def a_chunk(c, _, e=e, slot=slot):
                sl = pl.ds(c * CHUNK, CHUNK)
                o = ffn_chunk(x_ref, sl, slot)
                m = mloc_ref[sl, e:e + 1]
                out_ref[sl, :] = out_ref[sl, :] + m * o
                return _

            lax.fori_loop(0, N_CHUNKS, a_chunk, 0)
            start_load(e + 2 if e < 2 else e - 2)

        rd_x.wait()
        rd_m.wait()

        def b_chunk(c, carry):
            sl = pl.ds(c * CHUNK, CHUNK)
            acc = jnp.zeros((CHUNK, D), jnp.float32)
            for e in range(N_LOCAL):
                wait_load(e)
                op = ffn_chunk(rx, sl, e % 2)
                acc = acc + rmask[sl, e:e + 1] * op
                if e < 2:
                    start_load(e + 2)
                else:
                    @pl.when(c < N_CHUNKS - 1)
                    def _prefetch(e=e):
                        start_load(e - 2)
            pacc[sl, :] = acc.astype(jnp.bfloat16)
            back_rdma(c).start()
            return carry

        lax.fori_loop(0, N_CHUNKS, b_chunk, 0)

        def drain(c, _):
            back_rdma(c).wait()
            sl = pl.ds(c * CHUNK, CHUNK)
            out_ref[sl, :] = out_ref[sl, :] + rback[sl, :].astype(jnp.float32)
            return _

        lax.fori_loop(0, N_CHUNKS, drain, 0)

    return pl.pallas_call(
        body,
        out_shape=jax.ShapeDtypeStruct((T, D), jnp.float32),
        in_specs=[
            pl.BlockSpec(memory_space=pltpu.VMEM),
            pl.BlockSpec(memory_space=pltpu.VMEM),
            pl.BlockSpec(memory_space=pltpu.VMEM),
            pl.BlockSpec(memory_space=pltpu.HBM),
            pl.BlockSpec(memory_space=pltpu.HBM),
        ],
        out_specs=pl.BlockSpec(memory_space=pltpu.VMEM),
        scratch_shapes=[
            pltpu.VMEM((T, D), jnp.bfloat16),
            pltpu.VMEM((T, N_LOCAL), jnp.float32),
            pltpu.VMEM((T, D), jnp.bfloat16),
            pltpu.VMEM((T, D), jnp.bfloat16),
            pltpu.VMEM((2, D, F), jnp.bfloat16),
            pltpu.VMEM((2, F, D), jnp.bfloat16),
            pltpu.SemaphoreType.DMA((2, 2)),
            pltpu.SemaphoreType.DMA,
            pltpu.SemaphoreType.DMA,
            pltpu.SemaphoreType.DMA,
            pltpu.SemaphoreType.DMA,
            pltpu.SemaphoreType.DMA((N_CHUNKS,)),
            pltpu.SemaphoreType.DMA((N_CHUNKS,)),
        ],
        compiler_params=pltpu.CompilerParams(
            collective_id=0, vmem_limit_bytes=56 * 1024 * 1024),
    )(xb, mask_loc, mask_rem, w1b, w2b)
